# baseline (speedup 1.0000x reference)
import functools

import jax
import jax.numpy as jnp
from jax import lax
from jax.experimental import pallas as pl
from jax.experimental.pallas import tpu as pltpu
from jax.experimental.pallas import tpu_sc as plsc

B, T, D, K = 2, 196, 256, 512
N = B * T
NC, NS, L = 2, 16, 16
RPW = 16
NACT = (N + RPW - 1) // RPW


def _argmin_body(z_ref, cb_ref, idx_ref):
    z = z_ref[...]
    cbt = cb_ref[...].T
    scores = jnp.dot(z, cbt,
                     preferred_element_type=jnp.float32,
                     precision=lax.Precision.HIGHEST)
    cnorm = jnp.sum(cbt * cbt, axis=0)
    deltas = cnorm[None, :] - 2.0 * scores
    m = jnp.min(deltas, axis=1, keepdims=True)
    ids = lax.broadcasted_iota(jnp.int32, deltas.shape, 1)
    idx_ref[...] = jnp.min(jnp.where(deltas == m, ids, K), axis=1)


_tc_argmin = pl.pallas_call(
    _argmin_body,
    out_shape=jax.ShapeDtypeStruct((N,), jnp.int32),
)


def _sc_body(cb_hbm, idx_hbm, z_hbm, q_hbm, loss_hbm,
             idx_v, rows_v, z_v, loss_v, sem):
    wid = lax.axis_index("s") * NC + lax.axis_index("c")

    @pl.when(wid < NACT)
    def _():
        base = jnp.minimum(wid * RPW, N - RPW)
        pltpu.sync_copy(idx_hbm.at[pl.ds(base, RPW)], idx_v)
        gather = pltpu.async_copy(cb_hbm.at[idx_v], rows_v, sem)
        pltpu.sync_copy(z_hbm.at[pl.ds(base, RPW)], z_v)
        gather.wait()
        qcp = pltpu.async_copy(rows_v, q_hbm.at[pl.ds(base, RPW)], sem)

        def row_body(i, carry):
            for j in range(0, D, L):
                d = rows_v[i, pl.ds(j, L)] - z_v[i, pl.ds(j, L)]
                loss_v[i, pl.ds(j, L)] = d * d
            return carry

        lax.fori_loop(0, RPW, row_body, 0)
        lcp = pltpu.async_copy(loss_v, loss_hbm.at[pl.ds(base, RPW)], sem)
        qcp.wait()
        lcp.wait()


@functools.cache
def _sc_gather():
    return functools.partial(
        pl.kernel,
        mesh=plsc.VectorSubcoreMesh(core_axis_name="c", subcore_axis_name="s"),
        out_type=[jax.ShapeDtypeStruct((N, D), jnp.float32),
                  jax.ShapeDtypeStruct((N, D), jnp.float32)],
        scratch_types=[
            pltpu.VMEM((RPW,), jnp.int32),
            pltpu.VMEM((RPW, D), jnp.float32),
            pltpu.VMEM((RPW, D), jnp.float32),
            pltpu.VMEM((RPW, D), jnp.float32),
            pltpu.SemaphoreType.DMA,
        ],
    )(_sc_body)


def kernel(inputs, codebook):
    zf = inputs.transpose(1, 0, 2).reshape(N, D)
    idx = _tc_argmin(zf, codebook)
    q_rows, loss_rows = _sc_gather()(codebook, idx, zf)
    quantized = q_rows.reshape(T, B, D).transpose(1, 0, 2)[None]
    loss = loss_rows.reshape(T, B, D).transpose(1, 0, 2)[None]
    nn_idx = idx.reshape(T, B).transpose(1, 0)
    return (quantized, loss, nn_idx, codebook)


# speedup vs baseline: 1.0469x; 1.0469x over previous
import functools

import jax
import jax.numpy as jnp
from jax import lax
from jax.experimental import pallas as pl
from jax.experimental.pallas import tpu as pltpu
from jax.experimental.pallas import tpu_sc as plsc

B, T, D, K = 2, 196, 256, 512
N = B * T
NC, NS, L = 1, 16, 16
RPW = 32
NACT = (N + RPW - 1) // RPW


def _argmin_body(z_ref, cb_ref, idx_ref):
    z = z_ref[...]
    cbt = cb_ref[...].T
    scores = jnp.dot(z, cbt,
                     preferred_element_type=jnp.float32,
                     precision=lax.Precision.HIGHEST)
    cnorm = jnp.sum(cbt * cbt, axis=0)
    deltas = cnorm[None, :] - 2.0 * scores
    m = jnp.min(deltas, axis=1, keepdims=True)
    ids = lax.broadcasted_iota(jnp.int32, deltas.shape, 1)
    idx_ref[...] = jnp.min(jnp.where(deltas == m, ids, K), axis=1)


_tc_argmin = pl.pallas_call(
    _argmin_body,
    out_shape=jax.ShapeDtypeStruct((N,), jnp.int32),
)


def _sc_body(cb_hbm, idx_hbm, z_hbm, q_hbm, loss_hbm,
             idx_v, rows_v, z_v, loss_v, sem):
    wid = lax.axis_index("s") * NC + lax.axis_index("c")

    @pl.when(wid < NACT)
    def _():
        base = jnp.minimum(wid * RPW, N - RPW)
        pltpu.sync_copy(idx_hbm.at[pl.ds(base, RPW)], idx_v)
        gather = pltpu.async_copy(cb_hbm.at[idx_v], rows_v, sem)
        pltpu.sync_copy(z_hbm.at[pl.ds(base, RPW)], z_v)
        gather.wait()
        qcp = pltpu.async_copy(rows_v, q_hbm.at[pl.ds(base, RPW)], sem)

        def row_body(i, carry):
            for j in range(0, D, L):
                d = rows_v[i, pl.ds(j, L)] - z_v[i, pl.ds(j, L)]
                loss_v[i, pl.ds(j, L)] = d * d
            return carry

        lax.fori_loop(0, RPW, row_body, 0)
        lcp = pltpu.async_copy(loss_v, loss_hbm.at[pl.ds(base, RPW)], sem)
        qcp.wait()
        lcp.wait()


@functools.cache
def _sc_gather():
    return functools.partial(
        pl.kernel,
        mesh=plsc.VectorSubcoreMesh(core_axis_name="c", subcore_axis_name="s",
                                    num_cores=NC),
        out_type=[jax.ShapeDtypeStruct((N, D), jnp.float32),
                  jax.ShapeDtypeStruct((N, D), jnp.float32)],
        scratch_types=[
            pltpu.VMEM((RPW,), jnp.int32),
            pltpu.VMEM((RPW, D), jnp.float32),
            pltpu.VMEM((RPW, D), jnp.float32),
            pltpu.VMEM((RPW, D), jnp.float32),
            pltpu.SemaphoreType.DMA,
        ],
    )(_sc_body)


def kernel(inputs, codebook):
    zf = inputs.transpose(1, 0, 2).reshape(N, D)
    idx = _tc_argmin(zf, codebook)
    q_rows, loss_rows = _sc_gather()(codebook, idx, zf)
    quantized = q_rows.reshape(T, B, D).transpose(1, 0, 2)[None]
    loss = loss_rows.reshape(T, B, D).transpose(1, 0, 2)[None]
    nn_idx = idx.reshape(T, B).transpose(1, 0)
    return (quantized, loss, nn_idx, codebook)


# speedup vs baseline: 1.0471x; 1.0002x over previous
import functools

import jax
import jax.numpy as jnp
from jax import lax
from jax.experimental import pallas as pl
from jax.experimental.pallas import tpu as pltpu
from jax.experimental.pallas import tpu_sc as plsc

B, T, D, K = 2, 196, 256, 512
N = B * T
NC, NS, L = 1, 16, 16
RPW = 32
NACT = (N + RPW - 1) // RPW


def _argmin_body(z_ref, cb_ref, idx_ref):
    z = z_ref[...]
    cbt = cb_ref[...].T
    scores = jnp.dot(z, cbt,
                     preferred_element_type=jnp.float32,
                     precision=lax.Precision.HIGHEST)
    cnorm = jnp.sum(cbt * cbt, axis=0)
    deltas = cnorm[None, :] - 2.0 * scores
    m = jnp.min(deltas, axis=1, keepdims=True)
    ids = lax.broadcasted_iota(jnp.int32, deltas.shape, 1)
    idx_ref[...] = jnp.min(jnp.where(deltas == m, ids, K), axis=1)


_tc_argmin = pl.pallas_call(
    _argmin_body,
    out_shape=jax.ShapeDtypeStruct((N,), jnp.int32),
)


def _sc_body(cb_hbm, idx_hbm, z_hbm, q_hbm, loss_hbm,
             idx_v, rows_v, z_v, loss_v, sem):
    wid = lax.axis_index("s") * NC + lax.axis_index("c")

    @pl.when(wid < NACT)
    def _():
        base = jnp.minimum(wid * RPW, N - RPW)
        zcp = pltpu.async_copy(z_hbm.at[pl.ds(base, RPW)], z_v, sem)
        pltpu.sync_copy(idx_hbm.at[pl.ds(base, RPW)], idx_v)
        gather = pltpu.async_copy(cb_hbm.at[idx_v], rows_v, sem)
        zcp.wait()
        gather.wait()
        qcp = pltpu.async_copy(rows_v, q_hbm.at[pl.ds(base, RPW)], sem)

        def row_body(i, carry):
            for j in range(0, D, L):
                d = rows_v[i, pl.ds(j, L)] - z_v[i, pl.ds(j, L)]
                loss_v[i, pl.ds(j, L)] = d * d
            return carry

        lax.fori_loop(0, RPW, row_body, 0)
        lcp = pltpu.async_copy(loss_v, loss_hbm.at[pl.ds(base, RPW)], sem)
        qcp.wait()
        lcp.wait()


@functools.cache
def _sc_gather():
    return functools.partial(
        pl.kernel,
        mesh=plsc.VectorSubcoreMesh(core_axis_name="c", subcore_axis_name="s",
                                    num_cores=NC),
        out_type=[jax.ShapeDtypeStruct((N, D), jnp.float32),
                  jax.ShapeDtypeStruct((N, D), jnp.float32)],
        scratch_types=[
            pltpu.VMEM((RPW,), jnp.int32),
            pltpu.VMEM((RPW, D), jnp.float32),
            pltpu.VMEM((RPW, D), jnp.float32),
            pltpu.VMEM((RPW, D), jnp.float32),
            pltpu.SemaphoreType.DMA,
        ],
    )(_sc_body)


def kernel(inputs, codebook):
    zf = inputs.reshape(N, D)
    idx = _tc_argmin(zf, codebook)
    q_rows, loss_rows = _sc_gather()(codebook, idx, zf)
    quantized = q_rows.reshape(1, B, T, D)
    loss = loss_rows.reshape(1, B, T, D)
    nn_idx = idx.reshape(B, T)
    return (quantized, loss, nn_idx, codebook)
